# Initial kernel scaffold; baseline (speedup 1.0000x reference)
import jax
import jax.numpy as jnp
from jax.experimental import pallas as pl


def kernel(embeddings, attn, W, b):
    raise NotImplementedError("write your pallas kernel here")


# speedup vs baseline: 2.3209x; 2.3209x over previous
import functools

import jax
import jax.numpy as jnp
from jax.experimental import pallas as pl
from jax.experimental.pallas import tpu as pltpu

_RHO = 0.2
_TAU = 1.0
_TBLK = 1024


def _fwd_kernel(emb_ref, attn_ref, w_ref, b_ref, g_ref, z_ref, scores_ref,
                *, nT, T):
    t = pl.program_id(1)
    emb = emb_ref[0]
    w = w_ref[...]
    raw = jax.lax.dot_general(w, emb, (((1,), (1,)), ((), ())),
                              preferred_element_type=jnp.float32)
    scores_ref[:, pl.ds(t * _TBLK, _TBLK)] = raw

    @pl.when(t == nT - 1)
    def _tail():
        attn = attn_ref[0]
        bias = b_ref[0, 0]
        s = attn * scores_ref[...] + bias
        s = jnp.where(attn == 0.0, jnp.float32(-1e9), s)
        t_eff = jnp.sum(attn)
        kf = jnp.maximum(jnp.round(jnp.float32(_RHO) * t_eff), 1.0)
        s = s / jnp.float32(_TAU)
        e = jnp.exp(s - jnp.max(s))
        z = kf * (e / jnp.sum(e))

        bits = jax.lax.bitcast_convert_type(z, jnp.int32)
        key = jnp.where(bits < 0, bits ^ jnp.int32(0x7FFFFFFF), bits)
        ub = jax.lax.bitcast_convert_type(key, jnp.uint32) ^ jnp.uint32(0x80000000)
        ki = kf.astype(jnp.int32)

        def sbody(i, v):
            bit = jax.lax.shift_left(jnp.uint32(1), jnp.uint32(31) - i.astype(jnp.uint32))
            cand = v | bit
            cnt = jnp.sum((ub >= cand).astype(jnp.int32))
            return jnp.where(cnt >= ki, cand, v)

        v = jax.lax.fori_loop(0, 32, sbody, jnp.uint32(0))

        gt = ub > v
        eq = ub == v
        need = ki - jnp.sum(gt.astype(jnp.int32))
        idx = jax.lax.broadcasted_iota(jnp.int32, (1, T), 1)

        def ibody(i, lohi):
            lo, hi = lohi
            mid = (lo + hi) // 2
            cnt = jnp.sum((eq & (idx <= mid)).astype(jnp.int32))
            take = cnt >= need
            return (jnp.where(take, lo, mid + 1), jnp.where(take, mid, hi))

        lo, _ = jax.lax.fori_loop(0, 13, ibody,
                                  (jnp.int32(0), jnp.int32(T - 1)))
        h = (gt | (eq & (idx <= lo) & (need > 0))).astype(jnp.float32)
        g_ref[0] = h
        z_ref[0] = z


def kernel(embeddings, attn, W, b):
    B, T, D = embeddings.shape
    nT = T // _TBLK
    attn3 = attn.reshape(B, 1, T)
    b2 = b.reshape(1, 1)
    g3, z3 = pl.pallas_call(
        functools.partial(_fwd_kernel, nT=nT, T=T),
        grid=(B, nT),
        in_specs=[
            pl.BlockSpec((1, _TBLK, D), lambda bb, tt: (bb, tt, 0)),
            pl.BlockSpec((1, 1, T), lambda bb, tt: (bb, 0, 0)),
            pl.BlockSpec((1, D), lambda bb, tt: (0, 0)),
            pl.BlockSpec((1, 1), lambda bb, tt: (0, 0)),
        ],
        out_specs=[
            pl.BlockSpec((1, 1, T), lambda bb, tt: (bb, 0, 0)),
            pl.BlockSpec((1, 1, T), lambda bb, tt: (bb, 0, 0)),
        ],
        out_shape=[
            jax.ShapeDtypeStruct((B, 1, T), jnp.float32),
            jax.ShapeDtypeStruct((B, 1, T), jnp.float32),
        ],
        scratch_shapes=[pltpu.VMEM((1, T), jnp.float32)],
    )(embeddings, attn3, W, b2)
    return g3.reshape(B, T), z3.reshape(B, T)


# speedup vs baseline: 3.8289x; 1.6497x over previous
import functools

import jax
import jax.numpy as jnp
from jax.experimental import pallas as pl
from jax.experimental.pallas import tpu as pltpu

_RHO = 0.2
_TAU = 1.0
_TBLK = 1024


def _fwd_kernel(emb_ref, attn_ref, w_ref, b_ref, g_ref, z_ref, scores_ref,
                *, nB, nT, T):
    b = pl.program_id(0)
    t = pl.program_id(1)
    emb = emb_ref[0]
    w = w_ref[...]
    raw = jax.lax.dot_general(w, emb, (((1,), (1,)), ((), ())),
                              preferred_element_type=jnp.float32)
    scores_ref[pl.ds(b, 1), pl.ds(t * _TBLK, _TBLK)] = raw

    @pl.when((b == nB - 1) & (t == nT - 1))
    def _tail():
        attn = attn_ref[...]
        bias = b_ref[0, 0]
        s = attn * scores_ref[...] + bias
        s = jnp.where(attn == 0.0, jnp.float32(-1e9), s)
        t_eff = jnp.sum(attn, axis=1, keepdims=True)
        kf = jnp.maximum(jnp.round(jnp.float32(_RHO) * t_eff), 1.0)
        s = s / jnp.float32(_TAU)
        e = jnp.exp(s - jnp.max(s, axis=1, keepdims=True))
        z = kf * (e / jnp.sum(e, axis=1, keepdims=True))

        bits = jax.lax.bitcast_convert_type(z, jnp.int32)
        key = jnp.where(bits < 0, bits ^ jnp.int32(0x7FFFFFFF), bits)
        ub = jax.lax.bitcast_convert_type(key, jnp.uint32) ^ jnp.uint32(0x80000000)
        ki = kf.astype(jnp.int32)

        def sbody(i, v):
            bit = jax.lax.shift_left(jnp.uint32(1),
                                     jnp.uint32(31) - i.astype(jnp.uint32))
            cand = v | bit
            cnt = jnp.sum((ub >= cand).astype(jnp.int32), axis=1, keepdims=True)
            return jnp.where(cnt >= ki, cand, v)

        v = jax.lax.fori_loop(0, 32, sbody, jnp.zeros((nB, 1), jnp.uint32))

        gt = ub > v
        eq = ub == v
        need = ki - jnp.sum(gt.astype(jnp.int32), axis=1, keepdims=True)
        idx = jax.lax.broadcasted_iota(jnp.int32, (1, T), 1)

        def ibody(i, lohi):
            lo, hi = lohi
            mid = (lo + hi) // 2
            cnt = jnp.sum((eq & (idx <= mid)).astype(jnp.int32),
                          axis=1, keepdims=True)
            take = cnt >= need
            return (jnp.where(take, lo, mid + 1), jnp.where(take, mid, hi))

        lo0 = jnp.zeros((nB, 1), jnp.int32)
        hi0 = jnp.full((nB, 1), T - 1, jnp.int32)
        lo, _ = jax.lax.fori_loop(0, 13, ibody, (lo0, hi0))
        h = (gt | (eq & (idx <= lo) & (need > 0))).astype(jnp.float32)
        g_ref[...] = h
        z_ref[...] = z


def kernel(embeddings, attn, W, b):
    B, T, D = embeddings.shape
    nT = T // _TBLK
    b2 = b.reshape(1, 1)
    g, z = pl.pallas_call(
        functools.partial(_fwd_kernel, nB=B, nT=nT, T=T),
        grid=(B, nT),
        in_specs=[
            pl.BlockSpec((1, _TBLK, D), lambda bb, tt: (bb, tt, 0)),
            pl.BlockSpec((B, T), lambda bb, tt: (0, 0)),
            pl.BlockSpec((1, D), lambda bb, tt: (0, 0)),
            pl.BlockSpec((1, 1), lambda bb, tt: (0, 0)),
        ],
        out_specs=[
            pl.BlockSpec((B, T), lambda bb, tt: (0, 0)),
            pl.BlockSpec((B, T), lambda bb, tt: (0, 0)),
        ],
        out_shape=[
            jax.ShapeDtypeStruct((B, T), jnp.float32),
            jax.ShapeDtypeStruct((B, T), jnp.float32),
        ],
        scratch_shapes=[pltpu.VMEM((B, T), jnp.float32)],
    )(embeddings, attn, W, b2)
    return g, z


# speedup vs baseline: 4.6807x; 1.2225x over previous
import functools

import jax
import jax.numpy as jnp
from jax.experimental import pallas as pl
from jax.experimental.pallas import tpu as pltpu

_RHO = 0.2
_TAU = 1.0
_TBLK = 4096


def _fwd_kernel(emb_ref, attn_ref, w_ref, b_ref, g_ref, z_ref, scores_ref,
                *, nB, nT, T):
    b = pl.program_id(0)
    t = pl.program_id(1)
    emb = emb_ref[0]
    w = w_ref[...]
    raw = jax.lax.dot_general(w, emb, (((1,), (1,)), ((), ())),
                              preferred_element_type=jnp.float32)
    scores_ref[pl.ds(b, 1), pl.ds(t * _TBLK, _TBLK)] = raw

    @pl.when((b == nB - 1) & (t == nT - 1))
    def _tail():
        attn = attn_ref[...]
        bias = b_ref[0, 0]
        s = attn * scores_ref[...] + bias
        s = jnp.where(attn == 0.0, jnp.float32(-1e9), s)
        t_eff = jnp.sum(attn, axis=1, keepdims=True)
        kf = jnp.maximum(jnp.round(jnp.float32(_RHO) * t_eff), 1.0)
        s = s / jnp.float32(_TAU)
        e = jnp.exp(s - jnp.max(s, axis=1, keepdims=True))
        z = kf * (e / jnp.sum(e, axis=1, keepdims=True))

        bits = jax.lax.bitcast_convert_type(z, jnp.int32)
        key = jnp.where(bits < 0, bits ^ jnp.int32(0x7FFFFFFF), bits)
        ub = jax.lax.bitcast_convert_type(key, jnp.uint32) ^ jnp.uint32(0x80000000)
        ki = kf.astype(jnp.int32)

        def sbody(i, v):
            bit = jax.lax.shift_left(jnp.uint32(1),
                                     jnp.uint32(31) - i.astype(jnp.uint32))
            cand = v | bit
            cnt = jnp.sum((ub >= cand).astype(jnp.int32), axis=1, keepdims=True)
            return jnp.where(cnt >= ki, cand, v)

        v = jax.lax.fori_loop(0, 32, sbody, jnp.zeros((nB, 1), jnp.uint32))

        gt = ub > v
        eq = ub == v
        need = ki - jnp.sum(gt.astype(jnp.int32), axis=1, keepdims=True)
        idx = jax.lax.broadcasted_iota(jnp.int32, (1, T), 1)

        def ibody(i, lohi):
            lo, hi = lohi
            mid = (lo + hi) // 2
            cnt = jnp.sum((eq & (idx <= mid)).astype(jnp.int32),
                          axis=1, keepdims=True)
            take = cnt >= need
            return (jnp.where(take, lo, mid + 1), jnp.where(take, mid, hi))

        lo0 = jnp.zeros((nB, 1), jnp.int32)
        hi0 = jnp.full((nB, 1), T - 1, jnp.int32)
        lo, _ = jax.lax.fori_loop(0, 13, ibody, (lo0, hi0))
        h = (gt | (eq & (idx <= lo) & (need > 0))).astype(jnp.float32)
        g_ref[...] = h
        z_ref[...] = z


def kernel(embeddings, attn, W, b):
    B, T, D = embeddings.shape
    nT = T // _TBLK
    b2 = b.reshape(1, 1)
    g, z = pl.pallas_call(
        functools.partial(_fwd_kernel, nB=B, nT=nT, T=T),
        grid=(B, nT),
        in_specs=[
            pl.BlockSpec((1, _TBLK, D), lambda bb, tt: (bb, tt, 0)),
            pl.BlockSpec((B, T), lambda bb, tt: (0, 0)),
            pl.BlockSpec((1, D), lambda bb, tt: (0, 0)),
            pl.BlockSpec((1, 1), lambda bb, tt: (0, 0)),
        ],
        out_specs=[
            pl.BlockSpec((B, T), lambda bb, tt: (0, 0)),
            pl.BlockSpec((B, T), lambda bb, tt: (0, 0)),
        ],
        out_shape=[
            jax.ShapeDtypeStruct((B, T), jnp.float32),
            jax.ShapeDtypeStruct((B, T), jnp.float32),
        ],
        scratch_shapes=[pltpu.VMEM((B, T), jnp.float32)],
    )(embeddings, attn, W, b2)
    return g, z


# speedup vs baseline: 4.9822x; 1.0644x over previous
import functools

import jax
import jax.numpy as jnp
from jax.experimental import pallas as pl
from jax.experimental.pallas import tpu as pltpu

_RHO = 0.2
_TAU = 1.0
_TBLK = 4096


def _fwd_kernel(emb_ref, attn_ref, w_ref, b_ref, g_ref, z_ref, scores_ref,
                *, nB, nT, T):
    b = pl.program_id(0)
    t = pl.program_id(1)
    emb = emb_ref[0]
    w = w_ref[...]
    raw = jax.lax.dot_general(w, emb, (((1,), (1,)), ((), ())),
                              preferred_element_type=jnp.float32)
    scores_ref[pl.ds(b, 1), pl.ds(t * _TBLK, _TBLK)] = raw

    @pl.when((b == nB - 1) & (t == nT - 1))
    def _tail():
        attn = attn_ref[...]
        bias = b_ref[0, 0]
        s = attn * scores_ref[...] + bias
        s = jnp.where(attn == 0.0, jnp.float32(-1e9), s)
        t_eff = jnp.sum(attn, axis=1, keepdims=True)
        kf = jnp.maximum(jnp.round(jnp.float32(_RHO) * t_eff), 1.0)
        s = s / jnp.float32(_TAU)
        e = jnp.exp(s - jnp.max(s, axis=1, keepdims=True))
        z = kf * (e / jnp.sum(e, axis=1, keepdims=True))

        bits = jax.lax.bitcast_convert_type(z, jnp.int32)
        key = jnp.where(bits < 0, bits ^ jnp.int32(0x7FFFFFFF), bits)
        ub = jax.lax.bitcast_convert_type(key, jnp.uint32) ^ jnp.uint32(0x80000000)
        ki = kf.astype(jnp.int32)

        def _cnt_ge(c):
            return jnp.sum((ub >= c).astype(jnp.int32), axis=1, keepdims=True)

        def sbody(i, v):
            p_hi = (jnp.int32(15) - i) * 2 + 1
            b_hi = jax.lax.shift_left(jnp.uint32(1), p_hi.astype(jnp.uint32))
            b_lo = jax.lax.shift_left(jnp.uint32(1), (p_hi - 1).astype(jnp.uint32))
            c01 = v | b_lo
            c10 = v | b_hi
            c11 = c10 | b_lo
            n01, n10, n11 = _cnt_ge(c01), _cnt_ge(c10), _cnt_ge(c11)
            return jnp.where(n11 >= ki, c11,
                             jnp.where(n10 >= ki, c10,
                                       jnp.where(n01 >= ki, c01, v)))

        v = jax.lax.fori_loop(0, 16, sbody, jnp.zeros((nB, 1), jnp.uint32))

        gt = ub > v
        eq = ub == v
        need = ki - jnp.sum(gt.astype(jnp.int32), axis=1, keepdims=True)
        c_eq = jnp.sum(eq.astype(jnp.int32), axis=1, keepdims=True)
        idx = jax.lax.broadcasted_iota(jnp.int32, (1, T), 1)

        ties = jnp.logical_not(jnp.all(c_eq == need))

        def ibody(i, lohi):
            lo, hi = lohi
            mid = (lo + hi) // 2
            cnt = jnp.sum((eq & (idx <= mid)).astype(jnp.int32),
                          axis=1, keepdims=True)
            take = cnt >= need
            return (jnp.where(take, lo, mid + 1), jnp.where(take, mid, hi))

        lo0 = jnp.where(ties, 0, T - 1) * jnp.ones((nB, 1), jnp.int32)
        hi0 = jnp.full((nB, 1), T - 1, jnp.int32)
        trip = jnp.where(ties, 13, 0)
        lo, _ = jax.lax.fori_loop(0, trip, ibody, (lo0, hi0))
        h = (gt | (eq & (idx <= lo) & (need > 0))).astype(jnp.float32)
        g_ref[...] = h
        z_ref[...] = z


def kernel(embeddings, attn, W, b):
    B, T, D = embeddings.shape
    nT = T // _TBLK
    b2 = b.reshape(1, 1)
    g, z = pl.pallas_call(
        functools.partial(_fwd_kernel, nB=B, nT=nT, T=T),
        grid=(B, nT),
        in_specs=[
            pl.BlockSpec((1, _TBLK, D), lambda bb, tt: (bb, tt, 0)),
            pl.BlockSpec((B, T), lambda bb, tt: (0, 0)),
            pl.BlockSpec((1, D), lambda bb, tt: (0, 0)),
            pl.BlockSpec((1, 1), lambda bb, tt: (0, 0)),
        ],
        out_specs=[
            pl.BlockSpec((B, T), lambda bb, tt: (0, 0)),
            pl.BlockSpec((B, T), lambda bb, tt: (0, 0)),
        ],
        out_shape=[
            jax.ShapeDtypeStruct((B, T), jnp.float32),
            jax.ShapeDtypeStruct((B, T), jnp.float32),
        ],
        scratch_shapes=[pltpu.VMEM((B, T), jnp.float32)],
    )(embeddings, attn, W, b2)
    return g, z
